# baseline (speedup 1.0000x reference)
import functools

import jax
import jax.numpy as jnp
from jax import lax
from jax.experimental import pallas as pl
from jax.experimental.pallas import tpu as pltpu, tpu_sc as plsc

_info = plsc.get_sparse_core_info()
_NC, _NS = _info.num_cores, _info.num_subcores
_NW = _NC * _NS

_CHUNK = 64


def _embed_lookup(city_eo, pair_table):
    n_pairs = city_eo.shape[2]
    n_chunks = n_pairs // _CHUNK
    mesh = plsc.VectorSubcoreMesh(core_axis_name="c", subcore_axis_name="s")

    @functools.partial(
        pl.kernel,
        mesh=mesh,
        out_type=jax.ShapeDtypeStruct((_NW, n_chunks, _CHUNK, 128), jnp.float32),
        scratch_types=[
            pltpu.VMEM((2, n_pairs), jnp.int32),
            pltpu.VMEM((n_chunks, _CHUNK), jnp.int32),
            pltpu.VMEM((n_chunks, _CHUNK, 128), jnp.float32),
            pltpu.SemaphoreType.DMA,
            pltpu.SemaphoreType.DMA,
        ],
    )
    def k(ptab_hbm, idx_hbm, out_hbm, idx_v, pair_v, rows_v, sem, wsem):
        wid = lax.axis_index("s") * _NC + lax.axis_index("c")
        pltpu.sync_copy(idx_hbm.at[wid], idx_v)
        for g in range(n_pairs // 16):
            even = idx_v[0, pl.ds(g * 16, 16)]
            odd = idx_v[1, pl.ds(g * 16, 16)]
            r, c = divmod(g, _CHUNK // 16)
            pair_v[r, pl.ds(c * 16, 16)] = even * 5 + odd
        gathers = [
            pltpu.async_copy(ptab_hbm.at[pair_v.at[j]], rows_v.at[j], sem)
            for j in range(n_chunks)
        ]
        writes = []
        for j in range(n_chunks):
            gathers[j].wait()
            writes.append(
                pltpu.async_copy(rows_v.at[j], out_hbm.at[wid, j], wsem)
            )
        for w in writes:
            w.wait()

    return k(pair_table, city_eo)


def kernel(city, table):
    b = city.shape[0]
    d = table.shape[1]
    v = table.shape[0]
    left = jnp.repeat(table, v, axis=0)
    right = jnp.tile(table, (v, 1))
    pair_table = jnp.concatenate([left, right], axis=1)
    pair_table = jnp.pad(pair_table, ((0, 32 - v * v), (0, 0)))
    c3 = city.astype(jnp.int32).reshape(_NW, b // (2 * _NW), 2)
    city_eo = jnp.stack([c3[:, :, 0], c3[:, :, 1]], axis=1)
    out = _embed_lookup(city_eo, pair_table)
    return out.reshape(b, d)


# speedup vs baseline: 1.0257x; 1.0257x over previous
import functools

import jax
import jax.numpy as jnp
from jax import lax
from jax.experimental import pallas as pl
from jax.experimental.pallas import tpu as pltpu, tpu_sc as plsc

_info = plsc.get_sparse_core_info()
_NC, _NS = _info.num_cores, _info.num_subcores
_NW = _NC * _NS

_CHUNK = 128


def _embed_lookup(city_eo, pair_table):
    n_pairs = city_eo.shape[2]
    n_chunks = n_pairs // _CHUNK
    mesh = plsc.VectorSubcoreMesh(core_axis_name="c", subcore_axis_name="s")

    @functools.partial(
        pl.kernel,
        mesh=mesh,
        out_type=jax.ShapeDtypeStruct((_NW, n_chunks, _CHUNK, 128), jnp.float32),
        scratch_types=[
            pltpu.VMEM((2, n_pairs), jnp.int32),
            pltpu.VMEM((n_chunks, _CHUNK), jnp.int32),
            pltpu.VMEM((n_chunks, _CHUNK, 128), jnp.float32),
            pltpu.SemaphoreType.DMA,
            pltpu.SemaphoreType.DMA,
        ],
    )
    def k(ptab_hbm, idx_hbm, out_hbm, idx_v, pair_v, rows_v, sem, wsem):
        wid = lax.axis_index("s") * _NC + lax.axis_index("c")
        pltpu.sync_copy(idx_hbm.at[wid], idx_v)
        for g in range(n_pairs // 16):
            even = idx_v[0, pl.ds(g * 16, 16)]
            odd = idx_v[1, pl.ds(g * 16, 16)]
            r, c = divmod(g, _CHUNK // 16)
            pair_v[r, pl.ds(c * 16, 16)] = even * 5 + odd
        gathers = [
            pltpu.async_copy(ptab_hbm.at[pair_v.at[j]], rows_v.at[j], sem)
            for j in range(n_chunks)
        ]
        writes = []
        for j in range(n_chunks):
            gathers[j].wait()
            writes.append(
                pltpu.async_copy(rows_v.at[j], out_hbm.at[wid, j], wsem)
            )
        for w in writes:
            w.wait()

    return k(pair_table, city_eo)


def kernel(city, table):
    b = city.shape[0]
    d = table.shape[1]
    v = table.shape[0]
    left = jnp.repeat(table, v, axis=0)
    right = jnp.tile(table, (v, 1))
    pair_table = jnp.concatenate([left, right], axis=1)
    pair_table = jnp.pad(pair_table, ((0, 32 - v * v), (0, 0)))
    c3 = city.astype(jnp.int32).reshape(_NW, b // (2 * _NW), 2)
    city_eo = jnp.stack([c3[:, :, 0], c3[:, :, 1]], axis=1)
    out = _embed_lookup(city_eo, pair_table)
    return out.reshape(b, d)


# speedup vs baseline: 1.0769x; 1.0499x over previous
import functools

import jax
import jax.numpy as jnp
from jax import lax
from jax.experimental import pallas as pl
from jax.experimental.pallas import tpu as pltpu, tpu_sc as plsc

_info = plsc.get_sparse_core_info()
_NC, _NS = _info.num_cores, _info.num_subcores
_NW = _NC * _NS

_CHUNK = 256


def _embed_lookup(city_eo, pair_table):
    n_pairs = city_eo.shape[2]
    n_chunks = n_pairs // _CHUNK
    mesh = plsc.VectorSubcoreMesh(core_axis_name="c", subcore_axis_name="s")

    @functools.partial(
        pl.kernel,
        mesh=mesh,
        out_type=jax.ShapeDtypeStruct((_NW, n_chunks, _CHUNK, 128), jnp.float32),
        scratch_types=[
            pltpu.VMEM((2, n_pairs), jnp.int32),
            pltpu.VMEM((n_chunks, _CHUNK), jnp.int32),
            pltpu.VMEM((n_chunks, _CHUNK, 128), jnp.float32),
            pltpu.SemaphoreType.DMA,
            pltpu.SemaphoreType.DMA,
        ],
    )
    def k(ptab_hbm, idx_hbm, out_hbm, idx_v, pair_v, rows_v, sem, wsem):
        wid = lax.axis_index("s") * _NC + lax.axis_index("c")
        pltpu.sync_copy(idx_hbm.at[wid], idx_v)
        for g in range(n_pairs // 16):
            even = idx_v[0, pl.ds(g * 16, 16)]
            odd = idx_v[1, pl.ds(g * 16, 16)]
            r, c = divmod(g, _CHUNK // 16)
            pair_v[r, pl.ds(c * 16, 16)] = even * 5 + odd
        gathers = [
            pltpu.async_copy(ptab_hbm.at[pair_v.at[j]], rows_v.at[j], sem)
            for j in range(n_chunks)
        ]
        for g in gathers:
            g.wait()
        pltpu.sync_copy(rows_v, out_hbm.at[wid])

    return k(pair_table, city_eo)


def kernel(city, table):
    b = city.shape[0]
    d = table.shape[1]
    v = table.shape[0]
    left = jnp.repeat(table, v, axis=0)
    right = jnp.tile(table, (v, 1))
    pair_table = jnp.concatenate([left, right], axis=1)
    pair_table = jnp.pad(pair_table, ((0, 32 - v * v), (0, 0)))
    c3 = city.astype(jnp.int32).reshape(_NW, b // (2 * _NW), 2)
    city_eo = jnp.stack([c3[:, :, 0], c3[:, :, 1]], axis=1)
    out = _embed_lookup(city_eo, pair_table)
    return out.reshape(b, d)


# speedup vs baseline: 1.7751x; 1.6484x over previous
import functools

import jax
import jax.numpy as jnp
from jax import lax
from jax.experimental import pallas as pl
from jax.experimental.pallas import tpu as pltpu, tpu_sc as plsc

_info = plsc.get_sparse_core_info()
_NC, _NS = _info.num_cores, _info.num_subcores
_NW = _NC * _NS


def _embed_lookup(city2d, table):
    n_rows = city2d.shape[1]
    v, d = table.shape
    nc = d // 16
    mesh = plsc.VectorSubcoreMesh(core_axis_name="c", subcore_axis_name="s")

    @functools.partial(
        pl.kernel,
        mesh=mesh,
        out_type=jax.ShapeDtypeStruct((_NW * n_rows, d), jnp.float32),
        scratch_types=[
            pltpu.VMEM((n_rows,), jnp.int32),
            pltpu.VMEM((v, d), jnp.float32),
            pltpu.VMEM((n_rows, d), jnp.float32),
        ],
    )
    def k(tab_hbm, idx_hbm, out_hbm, idx_v, tab_v, rows_v):
        wid = lax.axis_index("s") * _NC + lax.axis_index("c")
        pltpu.sync_copy(tab_hbm, tab_v)
        pltpu.sync_copy(idx_hbm.at[wid], idx_v)

        def body(g, _):
            vec = idx_v[pl.ds(g * 16, 16)]
            for l in range(16):
                a = vec[l]
                i = g * 16 + l
                for c in range(nc):
                    rows_v[i, pl.ds(c * 16, 16)] = tab_v[a, pl.ds(c * 16, 16)]
            return 0

        lax.fori_loop(0, n_rows // 16, body, 0)
        pltpu.sync_copy(rows_v, out_hbm.at[pl.ds(wid * n_rows, n_rows)])

    return k(table, city2d)


def kernel(city, table):
    b = city.shape[0]
    city2d = city.astype(jnp.int32).reshape(_NW, b // _NW)
    return _embed_lookup(city2d, table)


# speedup vs baseline: 1.9893x; 1.1207x over previous
import functools

import jax
import jax.numpy as jnp
from jax import lax
from jax.experimental import pallas as pl
from jax.experimental.pallas import tpu as pltpu, tpu_sc as plsc

_info = plsc.get_sparse_core_info()
_NC, _NS = _info.num_cores, _info.num_subcores
_NW = _NC * _NS


def _embed_lookup(city2d, table):
    n_rows = city2d.shape[1]
    v, d = table.shape
    nc = d // 16
    mesh = plsc.VectorSubcoreMesh(core_axis_name="c", subcore_axis_name="s")

    @functools.partial(
        pl.kernel,
        mesh=mesh,
        out_type=jax.ShapeDtypeStruct((_NW * n_rows, d), jnp.float32),
        scratch_types=[
            pltpu.VMEM((n_rows,), jnp.int32),
            pltpu.VMEM((v, d), jnp.float32),
            pltpu.VMEM((n_rows, d), jnp.float32),
        ],
    )
    def k(tab_hbm, idx_hbm, out_hbm, idx_v, tab_v, rows_v):
        wid = lax.axis_index("s") * _NC + lax.axis_index("c")
        pltpu.sync_copy(tab_hbm, tab_v)
        pltpu.sync_copy(idx_hbm.at[wid], idx_v)

        @plsc.parallel_loop(0, n_rows // 16, unroll=4)
        def body(g):
            vec = idx_v[pl.ds(g * 16, 16)]
            for l in range(16):
                a = vec[l]
                i = g * 16 + l
                for c in range(nc):
                    rows_v[i, pl.ds(c * 16, 16)] = tab_v[a, pl.ds(c * 16, 16)]
        pltpu.sync_copy(rows_v, out_hbm.at[pl.ds(wid * n_rows, n_rows)])

    return k(table, city2d)


def kernel(city, table):
    b = city.shape[0]
    city2d = city.astype(jnp.int32).reshape(_NW, b // _NW)
    return _embed_lookup(city2d, table)
